# Initial kernel scaffold; baseline (speedup 1.0000x reference)
import jax
import jax.numpy as jnp
from jax.experimental import pallas as pl


def kernel(feature, edge_index, W1a, b1a, W2a, b2a, W1b, b1b, W2b, b2b):
    raise NotImplementedError("write your pallas kernel here")


# speedup vs baseline: 7.5624x; 7.5624x over previous
import functools

import jax
import jax.numpy as jnp
from jax import lax
from jax.experimental import pallas as pl
from jax.experimental.pallas import tpu as pltpu, tpu_sc as plsc

N, E, D = 10000, 320000, 128

NUM_CORES = 2
NUM_SUBCORES = 16
NUM_TILES = NUM_CORES * NUM_SUBCORES

CHUNK = 125
EDGES_PER_TILE = E // NUM_TILES
NCHUNK = EDGES_PER_TILE // CHUNK

N_PAD = 10240
ROWS_PER_TILE = N_PAD // NUM_SUBCORES


def _sc_agg_body(x_hbm, src_hbm, dst_hbm, z_hbm, out_hbm,
                 src_v, dst_v, gbuf, acc, gsem):
    cid = lax.axis_index("c")
    sid = lax.axis_index("s")
    blk = cid * NUM_SUBCORES + sid

    pltpu.sync_copy(src_hbm.at[blk], src_v)
    pltpu.sync_copy(dst_hbm.at[blk], dst_v)

    row0 = sid * ROWS_PER_TILE
    pltpu.sync_copy(z_hbm, acc.at[pl.ds(row0, ROWS_PER_TILE)])
    plsc.subcore_barrier()

    def body(ch, _):
        pltpu.async_copy(x_hbm.at[src_v.at[ch]], gbuf, gsem).wait()
        pltpu.sync_copy(gbuf, acc.at[dst_v.at[ch]], add=True)
        return _

    lax.fori_loop(0, NCHUNK, body, None)
    plsc.subcore_barrier()

    pltpu.sync_copy(acc.at[pl.ds(row0, ROWS_PER_TILE)],
                    out_hbm.at[cid, pl.ds(row0, ROWS_PER_TILE)])


_sc_agg = pl.kernel(
    _sc_agg_body,
    out_type=jax.ShapeDtypeStruct((NUM_CORES, N_PAD, D), jnp.float32),
    mesh=plsc.VectorSubcoreMesh(core_axis_name="c", subcore_axis_name="s"),
    scratch_types=[
        pltpu.VMEM((NCHUNK, CHUNK), jnp.int32),
        pltpu.VMEM((NCHUNK, CHUNK), jnp.int32),
        pltpu.VMEM((CHUNK, D), jnp.float32),
        pltpu.VMEM_SHARED((N_PAD, D), jnp.float32),
        pltpu.SemaphoreType.DMA,
    ],
)


def _mlp_block(x_ref, p_ref, w1_ref, b1_ref, w2_ref, b2_ref, o_ref, *, elu):
    h = x_ref[...] + p_ref[0] + p_ref[1]
    h = jnp.maximum(jnp.dot(h, w1_ref[...],
                            preferred_element_type=jnp.float32) + b1_ref[...],
                    0.0)
    y = jnp.dot(h, w2_ref[...], preferred_element_type=jnp.float32) + b2_ref[...]
    if elu:
        y = jnp.where(y > 0.0, y, jnp.exp(jnp.minimum(y, 0.0)) - 1.0)
    o_ref[...] = y


_ROW_BLK = 1000


def _mlp(x, partials, w1, b1, w2, b2, elu):
    grid = (N // _ROW_BLK,)
    return pl.pallas_call(
        functools.partial(_mlp_block, elu=elu),
        grid=grid,
        in_specs=[
            pl.BlockSpec((_ROW_BLK, D), lambda i: (i, 0)),
            pl.BlockSpec((NUM_CORES, _ROW_BLK, D), lambda i: (0, i, 0)),
            pl.BlockSpec((D, D), lambda i: (0, 0)),
            pl.BlockSpec((1, D), lambda i: (0, 0)),
            pl.BlockSpec((D, D), lambda i: (0, 0)),
            pl.BlockSpec((1, D), lambda i: (0, 0)),
        ],
        out_specs=pl.BlockSpec((_ROW_BLK, D), lambda i: (i, 0)),
        out_shape=jax.ShapeDtypeStruct((N, D), jnp.float32),
    )(x, partials, w1, b1.reshape(1, D), w2, b2.reshape(1, D))


def kernel(feature, edge_index, W1a, b1a, W2a, b2a, W1b, b1b, W2b, b2b):
    src = edge_index[0].reshape(NUM_TILES, NCHUNK, CHUNK)
    dst = edge_index[1].reshape(NUM_TILES, NCHUNK, CHUNK)
    zeros = jnp.zeros((ROWS_PER_TILE, D), jnp.float32)

    p1 = _sc_agg(feature, src, dst, zeros)
    x1 = _mlp(feature, p1, W1a, b1a, W2a, b2a, elu=True)
    p2 = _sc_agg(x1, src, dst, zeros)
    return _mlp(x1, p2, W1b, b1b, W2b, b2b, elu=False)
